# baseline (speedup 1.0000x reference)
import functools

import jax
import jax.numpy as jnp
from jax import lax
from jax.experimental import pallas as pl
from jax.experimental.pallas import tpu as pltpu
from jax.experimental.pallas import tpu_sc as plsc

B = 16384
F = 26
EMB = 16
ND = 8
OUT = 64
VOCAB = 1000000

NC = 2
NS = 16
NW = NC * NS
LANES = 16

N_FLAT = B * F
N_PER_W = N_FLAT // NW
PERIOD = 208
N_CHUNK = 16
G = N_PER_W // N_CHUNK


TILE_COLS = 128
N_TILES_FULL = VOCAB // TILE_COLS
TAIL_COL0 = N_TILES_FULL * TILE_COLS
BLK_PER_W = N_TILES_FULL // NW
BLK_REM = N_TILES_FULL - BLK_PER_W * NW


def _sc_transpose_body(tT_hbm, tail_hbm, out_hbm, stage, ostage, isem):
    wid = lax.axis_index("s") * NC + lax.axis_index("c")
    nblk = BLK_PER_W + (wid < BLK_REM).astype(jnp.int32)
    start = wid * BLK_PER_W + jnp.minimum(wid, BLK_REM)
    lane = lax.iota(jnp.int32, 16)
    i0 = lane // 8
    i1 = lane % 8

    def do_block(col0, nrows):
        c1 = pltpu.async_copy(
            tT_hbm.at[pl.ds(0, 8), pl.ds(col0, TILE_COLS)], stage.at[0], isem)
        c2 = pltpu.async_copy(
            tT_hbm.at[pl.ds(8, 8), pl.ds(col0, TILE_COLS)], stage.at[1], isem)
        c1.wait()
        c2.wait()

        def row(l, carry):
            g = plsc.load_gather(stage, [i0, i1, jnp.full((16,), l, jnp.int32)])
            ostage[pl.ds(l * 16, 16)] = g
            return carry

        lax.fori_loop(0, nrows, row, 0)

    def blk(i, carry):
        col0 = (start + i) * TILE_COLS
        do_block(col0, TILE_COLS)
        pltpu.sync_copy(ostage, out_hbm.at[pl.ds(col0 * EMB, TILE_COLS * EMB)])
        return carry

    lax.fori_loop(0, nblk, blk, 0)

    @pl.when(wid == NW - 1)
    def _tail():
        c1 = pltpu.async_copy(tail_hbm.at[pl.ds(0, 8), :], stage.at[0], isem)
        c2 = pltpu.async_copy(tail_hbm.at[pl.ds(8, 8), :], stage.at[1], isem)
        c1.wait()
        c2.wait()

        def row(l, carry):
            g = plsc.load_gather(stage, [i0, i1, jnp.full((16,), l, jnp.int32)])
            ostage[pl.ds(l * 16, 16)] = g
            return carry

        lax.fori_loop(0, VOCAB - TAIL_COL0, row, 0)
        pltpu.sync_copy(
            ostage.at[pl.ds(0, (VOCAB - TAIL_COL0) * EMB)],
            out_hbm.at[pl.ds(TAIL_COL0 * EMB, (VOCAB - TAIL_COL0) * EMB)])


@jax.jit
def _sc_transpose(tableT, tail_pad):
    mesh = plsc.VectorSubcoreMesh(core_axis_name="c", subcore_axis_name="s")
    f = pl.kernel(
        _sc_transpose_body,
        out_type=jax.ShapeDtypeStruct((VOCAB * EMB,), jnp.float32),
        mesh=mesh,
        scratch_types=[
            pltpu.VMEM((2, 8, TILE_COLS), jnp.float32),
            pltpu.VMEM((TILE_COLS * EMB,), jnp.float32),
            pltpu.SemaphoreType.DMA,
        ],
        compiler_params=pltpu.CompilerParams(
            use_tc_tiling_on_sc=True, needs_layout_passes=False),
    )
    return f(tableT, tail_pad)


def _sc_gather_body(ids_hbm, pat_hbm, table_hbm, out_hbm, idx_v, pat_v, rows_v, gsem):
    wid = lax.axis_index("s") * NC + lax.axis_index("c")
    base = wid * N_PER_W
    pltpu.sync_copy(ids_hbm.at[pl.ds(base, N_PER_W)], idx_v)
    pltpu.sync_copy(pat_hbm, pat_v)

    pats = tuple(pat_v[pl.ds(j * LANES, LANES)] for j in range(PERIOD // LANES))

    def tbody(t, pats):
        p0 = t * PERIOD
        for j in range(PERIOD // LANES):
            sl = pl.ds(p0 + j * LANES, LANES)
            idx_v[sl] = idx_v[sl] + pats[j]
        return pats

    lax.fori_loop(0, N_PER_W // PERIOD, tbody, pats)

    def gbody(c, carry):
        off = c * G
        pltpu.async_copy(table_hbm.at[idx_v.at[pl.ds(off, G)]], rows_v, gsem).wait()
        pltpu.sync_copy(rows_v, out_hbm.at[pl.ds(base + off, G)])
        return carry

    lax.fori_loop(0, N_CHUNK, gbody, 0)


@jax.jit
def _sc_gather(ids_flat, pattern, table):
    mesh = plsc.VectorSubcoreMesh(core_axis_name="c", subcore_axis_name="s")
    f = pl.kernel(
        _sc_gather_body,
        out_type=jax.ShapeDtypeStruct((N_FLAT, EMB), jnp.float32),
        mesh=mesh,
        scratch_types=[
            pltpu.VMEM((N_PER_W,), jnp.int32),
            pltpu.VMEM((PERIOD,), jnp.int32),
            pltpu.VMEM((G, EMB), jnp.float32),
            pltpu.SemaphoreType.DMA,
        ],
        compiler_params=pltpu.CompilerParams(use_tc_tiling_on_sc=False),
    )
    return f(ids_flat, pattern, table)


def _mlp_body(dense_ref, emb_ref, wd_ref, wc_ref, b_ref, out_ref):
    acc = jnp.dot(dense_ref[...], wd_ref[...], preferred_element_type=jnp.float32)
    acc = acc + jnp.dot(emb_ref[...], wc_ref[...], preferred_element_type=jnp.float32)
    acc = acc + b_ref[...]
    out_ref[...] = 0.5 * acc * (1.0 + lax.erf(acc * (2.0 ** -0.5)))


@functools.partial(jax.jit, static_argnames=("bm",))
def _mlp(dense, emb, wd, wc, b, bm=2048):
    grid = (B // bm,)
    return pl.pallas_call(
        _mlp_body,
        grid=grid,
        in_specs=[
            pl.BlockSpec((bm, ND), lambda i: (i, 0)),
            pl.BlockSpec((bm, F * EMB), lambda i: (i, 0)),
            pl.BlockSpec((ND, OUT), lambda i: (0, 0)),
            pl.BlockSpec((F * EMB, OUT), lambda i: (0, 0)),
            pl.BlockSpec((1, OUT), lambda i: (0, 0)),
        ],
        out_specs=pl.BlockSpec((bm, OUT), lambda i: (i, 0)),
        out_shape=jax.ShapeDtypeStruct((B, OUT), jnp.float32),
    )(dense, emb, wd, wc, b)


def kernel(dense_features, categorical_ids, field_offsets, table, W, b):
    ids_flat = categorical_ids.reshape(-1)
    pattern = jnp.tile(field_offsets, PERIOD // F)
    tableT = table.T
    tail_pad = jnp.pad(
        tableT[:, TAIL_COL0:], ((0, 0), (0, TILE_COLS - (VOCAB - TAIL_COL0))))
    table_lin = _sc_transpose(tableT, tail_pad).reshape(VOCAB, EMB)
    emb = _sc_gather(ids_flat, pattern, table_lin)
    emb2d = emb.reshape(B, F * EMB)
    wd = W[:ND]
    wc = W[ND:]
    return _mlp(dense_features, emb2d, wd, wc, b.reshape(1, OUT))


# speedup vs baseline: 1.3322x; 1.3322x over previous
import functools

import jax
import jax.numpy as jnp
from jax import lax
from jax.experimental import pallas as pl
from jax.experimental.pallas import tpu as pltpu
from jax.experimental.pallas import tpu_sc as plsc

B = 16384
F = 26
EMB = 16
ND = 8
OUT = 64
VOCAB = 1000000

NC = 2
NS = 16
NW = NC * NS
LANES = 16

N_FLAT = B * F
N_PER_W = N_FLAT // NW
PERIOD = 208
N_CHUNK = 16
G = N_PER_W // N_CHUNK


TILE_COLS = 128
N_TILES_FULL = VOCAB // TILE_COLS
TAIL_COL0 = N_TILES_FULL * TILE_COLS
BLK_PER_W = N_TILES_FULL // NW
BLK_REM = N_TILES_FULL - BLK_PER_W * NW


SBW = 512
CH = 61
T_PER_W = CH * (SBW // TILE_COLS)
EXTRA_T0 = NW * T_PER_W


def _sc_transpose_body(tT_hbm, tail_hbm, out_hbm, stage, obuf,
                       isem0, isem1, osem0, osem1):
    wid = lax.axis_index("s") * NC + lax.axis_index("c")
    start_col = wid * (T_PER_W * TILE_COLS)
    lane = lax.iota(jnp.int32, 16)
    i0 = lane // 8
    i1 = lane % 8
    isems = (isem0, isem1)
    osems = (osem0, osem1)

    def fire_in(i, b):
        col0 = start_col + i * SBW
        pltpu.async_copy(tT_hbm.at[pl.ds(0, 8), pl.ds(col0, SBW)],
                         stage.at[b, 0], isems[b])
        pltpu.async_copy(tT_hbm.at[pl.ds(8, 8), pl.ds(col0, SBW)],
                         stage.at[b, 1], isems[b])

    def wait_in(i, b):
        col0 = start_col + i * SBW
        pltpu.make_async_copy(tT_hbm.at[pl.ds(0, 8), pl.ds(col0, SBW)],
                              stage.at[b, 0], isems[b]).wait()
        pltpu.make_async_copy(tT_hbm.at[pl.ds(8, 8), pl.ds(col0, SBW)],
                              stage.at[b, 1], isems[b]).wait()

    def out_slice(i):
        return out_hbm.at[pl.ds((start_col + i * SBW) * EMB, SBW * EMB)]

    def extract(b, nrows_div8):
        def rowblk(l8, carry):
            for u in range(8):
                l = l8 * 8 + u
                g = plsc.load_gather(
                    stage.at[b], [i0, i1, jnp.full((16,), l, jnp.int32)])
                obuf[b, pl.ds(l * 16, 16)] = g
            return carry
        lax.fori_loop(0, nrows_div8, rowblk, 0)

    def chunk_step(i, b):
        wait_in(i, b)

        @pl.when(i + 1 < CH)
        def _():
            fire_in(i + 1, b ^ 1)

        @pl.when(i >= 2)
        def _():
            pltpu.make_async_copy(obuf.at[b], out_slice(i - 2), osems[b]).wait()

        extract(b, SBW // 8)
        pltpu.async_copy(obuf.at[b], out_slice(i), osems[b])

    fire_in(0, 0)

    def pair(k, carry):
        chunk_step(k * 2, 0)

        @pl.when(k * 2 + 1 < CH)
        def _():
            chunk_step(k * 2 + 1, 1)
        return carry

    lax.fori_loop(0, (CH + 1) // 2, pair, 0)
    pltpu.make_async_copy(obuf.at[1], out_slice(CH - 2), osems[1]).wait()
    pltpu.make_async_copy(obuf.at[0], out_slice(CH - 1), osems[0]).wait()

    @pl.when(wid < N_TILES_FULL - EXTRA_T0)
    def _extra():
        col0 = (EXTRA_T0 + wid) * TILE_COLS
        pltpu.async_copy(tT_hbm.at[pl.ds(0, 8), pl.ds(col0, TILE_COLS)],
                         stage.at[0, 0, :, pl.ds(0, TILE_COLS)], isem0)
        pltpu.async_copy(tT_hbm.at[pl.ds(8, 8), pl.ds(col0, TILE_COLS)],
                         stage.at[0, 1, :, pl.ds(0, TILE_COLS)], isem0)
        pltpu.make_async_copy(tT_hbm.at[pl.ds(0, 8), pl.ds(col0, TILE_COLS)],
                              stage.at[0, 0, :, pl.ds(0, TILE_COLS)], isem0).wait()
        pltpu.make_async_copy(tT_hbm.at[pl.ds(8, 8), pl.ds(col0, TILE_COLS)],
                              stage.at[0, 1, :, pl.ds(0, TILE_COLS)], isem0).wait()
        extract(0, TILE_COLS // 8)
        pltpu.sync_copy(obuf.at[0, pl.ds(0, TILE_COLS * EMB)],
                        out_hbm.at[pl.ds(col0 * EMB, TILE_COLS * EMB)])

    @pl.when(wid == NW - 1)
    def _tail():
        pltpu.async_copy(tail_hbm.at[pl.ds(0, 8), :],
                         stage.at[0, 0, :, pl.ds(0, TILE_COLS)], isem0)
        pltpu.async_copy(tail_hbm.at[pl.ds(8, 8), :],
                         stage.at[0, 1, :, pl.ds(0, TILE_COLS)], isem0)
        pltpu.make_async_copy(tail_hbm.at[pl.ds(0, 8), :],
                              stage.at[0, 0, :, pl.ds(0, TILE_COLS)], isem0).wait()
        pltpu.make_async_copy(tail_hbm.at[pl.ds(8, 8), :],
                              stage.at[0, 1, :, pl.ds(0, TILE_COLS)], isem0).wait()
        extract(0, (VOCAB - TAIL_COL0) // 8)
        pltpu.sync_copy(
            obuf.at[0, pl.ds(0, (VOCAB - TAIL_COL0) * EMB)],
            out_hbm.at[pl.ds(TAIL_COL0 * EMB, (VOCAB - TAIL_COL0) * EMB)])


@jax.jit
def _sc_transpose(tableT, tail_pad):
    mesh = plsc.VectorSubcoreMesh(core_axis_name="c", subcore_axis_name="s")
    f = pl.kernel(
        _sc_transpose_body,
        out_type=jax.ShapeDtypeStruct((VOCAB * EMB,), jnp.float32),
        mesh=mesh,
        scratch_types=[
            pltpu.VMEM((2, 2, 8, SBW), jnp.float32),
            pltpu.VMEM((2, SBW * EMB), jnp.float32),
            pltpu.SemaphoreType.DMA,
            pltpu.SemaphoreType.DMA,
            pltpu.SemaphoreType.DMA,
            pltpu.SemaphoreType.DMA,
        ],
        compiler_params=pltpu.CompilerParams(
            use_tc_tiling_on_sc=True, needs_layout_passes=False),
    )
    return f(tableT, tail_pad)


def _sc_gather_body(ids_hbm, pat_hbm, table_hbm, out_hbm, idx_v, pat_v, rows_v, gsem):
    wid = lax.axis_index("s") * NC + lax.axis_index("c")
    base = wid * N_PER_W
    pltpu.sync_copy(ids_hbm.at[pl.ds(base, N_PER_W)], idx_v)
    pltpu.sync_copy(pat_hbm, pat_v)

    pats = tuple(pat_v[pl.ds(j * LANES, LANES)] for j in range(PERIOD // LANES))

    def tbody(t, pats):
        p0 = t * PERIOD
        for j in range(PERIOD // LANES):
            sl = pl.ds(p0 + j * LANES, LANES)
            idx_v[sl] = idx_v[sl] + pats[j]
        return pats

    lax.fori_loop(0, N_PER_W // PERIOD, tbody, pats)

    def gbody(c, carry):
        off = c * G
        pltpu.async_copy(table_hbm.at[idx_v.at[pl.ds(off, G)]], rows_v, gsem).wait()
        pltpu.sync_copy(rows_v, out_hbm.at[pl.ds(base + off, G)])
        return carry

    lax.fori_loop(0, N_CHUNK, gbody, 0)


@jax.jit
def _sc_gather(ids_flat, pattern, table):
    mesh = plsc.VectorSubcoreMesh(core_axis_name="c", subcore_axis_name="s")
    f = pl.kernel(
        _sc_gather_body,
        out_type=jax.ShapeDtypeStruct((N_FLAT, EMB), jnp.float32),
        mesh=mesh,
        scratch_types=[
            pltpu.VMEM((N_PER_W,), jnp.int32),
            pltpu.VMEM((PERIOD,), jnp.int32),
            pltpu.VMEM((G, EMB), jnp.float32),
            pltpu.SemaphoreType.DMA,
        ],
        compiler_params=pltpu.CompilerParams(use_tc_tiling_on_sc=False),
    )
    return f(ids_flat, pattern, table)


def _mlp_body(dense_ref, emb_ref, wd_ref, wc_ref, b_ref, out_ref):
    acc = jnp.dot(dense_ref[...], wd_ref[...], preferred_element_type=jnp.float32)
    acc = acc + jnp.dot(emb_ref[...], wc_ref[...], preferred_element_type=jnp.float32)
    acc = acc + b_ref[...]
    out_ref[...] = 0.5 * acc * (1.0 + lax.erf(acc * (2.0 ** -0.5)))


@functools.partial(jax.jit, static_argnames=("bm",))
def _mlp(dense, emb, wd, wc, b, bm=2048):
    grid = (B // bm,)
    return pl.pallas_call(
        _mlp_body,
        grid=grid,
        in_specs=[
            pl.BlockSpec((bm, ND), lambda i: (i, 0)),
            pl.BlockSpec((bm, F * EMB), lambda i: (i, 0)),
            pl.BlockSpec((ND, OUT), lambda i: (0, 0)),
            pl.BlockSpec((F * EMB, OUT), lambda i: (0, 0)),
            pl.BlockSpec((1, OUT), lambda i: (0, 0)),
        ],
        out_specs=pl.BlockSpec((bm, OUT), lambda i: (i, 0)),
        out_shape=jax.ShapeDtypeStruct((B, OUT), jnp.float32),
    )(dense, emb, wd, wc, b)


def kernel(dense_features, categorical_ids, field_offsets, table, W, b):
    ids_flat = categorical_ids.reshape(-1)
    pattern = jnp.tile(field_offsets, PERIOD // F)
    tableT = table.T
    tail_pad = jnp.pad(
        tableT[:, TAIL_COL0:], ((0, 0), (0, TILE_COLS - (VOCAB - TAIL_COL0))))
    table_lin = _sc_transpose(tableT, tail_pad).reshape(VOCAB, EMB)
    emb = _sc_gather(ids_flat, pattern, table_lin)
    emb2d = emb.reshape(B, F * EMB)
    wd = W[:ND]
    wc = W[ND:]
    return _mlp(dense_features, emb2d, wd, wc, b.reshape(1, OUT))
